# baseline (speedup 1.0000x reference)
import jax
import jax.numpy as jnp
from jax.experimental import pallas as pl


def _bcast_body(emb_ref, out_ref):
    s = pl.program_id(1)
    blk = out_ref.shape[1]
    out_ref[...] = emb_ref[pl.ds(s * blk, blk), :][None]


def kernel(sequence, embeddings):
    batch, seq_len, feat = sequence.shape

    blk = 1024
    while seq_len % blk != 0:
        blk //= 2
    nsb = seq_len // blk

    return pl.pallas_call(
        _bcast_body,
        grid=(batch, nsb),
        in_specs=[pl.BlockSpec((seq_len, feat), lambda b, s: (0, 0))],
        out_specs=pl.BlockSpec((1, blk, feat), lambda b, s: (b, s, 0)),
        out_shape=jax.ShapeDtypeStruct((batch, seq_len, feat), sequence.dtype),
    )(embeddings)
